# baseline (speedup 1.0000x reference)
import jax
import jax.numpy as jnp
from jax.experimental import pallas as pl
from jax.experimental.pallas import tpu as pltpu

N = 10000
D = 128
BM = 200


def _matmul_block(adj_ref, x_ref, out_ref):
    a = adj_ref[...].astype(jnp.bfloat16)
    out_ref[...] = jnp.dot(
        a, x_ref[...], preferred_element_type=jnp.float32
    )


def kernel(x, adj):
    x = x.astype(jnp.bfloat16)
    grid = (N // BM,)
    return pl.pallas_call(
        _matmul_block,
        grid=grid,
        in_specs=[
            pl.BlockSpec((BM, N), lambda i: (i, 0)),
            pl.BlockSpec((N, D), lambda i: (0, 0)),
        ],
        out_specs=pl.BlockSpec((BM, D), lambda i: (i, 0)),
        out_shape=jax.ShapeDtypeStruct((N, D), jnp.float32),
        compiler_params=pltpu.CompilerParams(
            dimension_semantics=("parallel",),
        ),
    )(adj, x)


# speedup vs baseline: 1.0510x; 1.0510x over previous
import jax
import jax.numpy as jnp
from jax.experimental import pallas as pl
from jax.experimental.pallas import tpu as pltpu

N = 10000
D = 128
BM = 200


def _matmul_block(adj_ref, x_ref, out_ref):
    out_ref[...] = jnp.dot(
        adj_ref[...], x_ref[...], preferred_element_type=jnp.float32
    )


def kernel(x, adj):
    grid = (N // BM,)
    return pl.pallas_call(
        _matmul_block,
        grid=grid,
        in_specs=[
            pl.BlockSpec((BM, N), lambda i: (i, 0)),
            pl.BlockSpec((N, D), lambda i: (0, 0)),
        ],
        out_specs=pl.BlockSpec((BM, D), lambda i: (i, 0)),
        out_shape=jax.ShapeDtypeStruct((N, D), jnp.float32),
        compiler_params=pltpu.CompilerParams(
            dimension_semantics=("parallel",),
        ),
    )(adj, x)
